# baseline (speedup 1.0000x reference)
import functools

import jax
import jax.numpy as jnp
from jax import lax
from jax.experimental import pallas as pl
from jax.experimental.pallas import tpu as pltpu
from jax.experimental.pallas import tpu_sc as plsc

NC = 2
NS = 16
LANES = 16

CHUNK = 64


def _sc_partial_sums(x, scope1d, sc_rows, d, nbags):
    nworkers = NC * NS
    rows_per_worker = sc_rows // nworkers
    nchunks = rows_per_worker // CHUNK
    nj = d // LANES
    mesh = plsc.VectorSubcoreMesh(core_axis_name="c", subcore_axis_name="s")

    @functools.partial(
        pl.kernel,
        out_type=jax.ShapeDtypeStruct((nworkers, nbags, d), jnp.float32),
        mesh=mesh,
        scratch_types=[
            pltpu.VMEM((2, CHUNK, d), jnp.float32),
            pltpu.VMEM((nbags + 8, ), jnp.int32),
            pltpu.VMEM((nbags, d), jnp.float32),
            pltpu.SemaphoreType.DMA,
            pltpu.SemaphoreType.DMA,
            pltpu.SemaphoreType.DMA,
            pltpu.SemaphoreType.DMA,
        ],
    )
    def body(x_hbm, cuts_hbm, out_hbm, xbuf, cutsv, acc,
             sem0, sem1, sem0b, sem1b):
        c = lax.axis_index("c")
        s = lax.axis_index("s")
        wid = c * NS + s
        base = wid * rows_per_worker

        pltpu.make_async_copy(
            x_hbm.at[pl.ds(base, CHUNK)], xbuf.at[0], sem0).start()
        pltpu.sync_copy(cuts_hbm, cutsv.at[pl.ds(0, nbags + 1)])

        def zrow(i, carry):
            for j in range(nj):
                acc[i, pl.ds(j * LANES, LANES)] = jnp.zeros(
                    (LANES,), jnp.float32)
            return carry
        lax.fori_loop(0, nbags, zrow, 0)

        def start_chunk(g0, slot_idx, s_a, s_b):
            del s_b
            pltpu.make_async_copy(
                x_hbm.at[pl.ds(g0, CHUNK)], xbuf.at[slot_idx], s_a).start()

        def wait_chunk(slot_idx, s_a, s_b):
            del s_b
            pltpu.make_async_copy(
                x_hbm.at[pl.ds(base, CHUNK)], xbuf.at[slot_idx], s_a).wait()

        cuts_vec = cutsv[pl.ds(1, nbags)]
        cut_s = [cuts_vec[i] for i in range(nbags)]

        def bag_of(row):
            b = jnp.int32(0)
            for i in range(nbags):
                b = b + jnp.where(cut_s[i] <= row, 1, 0).astype(jnp.int32)
            return b

        def cut_at(b):
            cn = jnp.int32(0)
            for i in range(nbags):
                cn = cn + jnp.where(b == i, cut_s[i], 0).astype(jnp.int32)
            return cn

        GRP = 8

        def chunk_body(g, carry):
            nxt = g + 1
            even_nxt = lax.rem(nxt, 2) == 0

            @pl.when((nxt < nchunks) & even_nxt)
            def _start_even():
                start_chunk(base + nxt * CHUNK, 0, sem0, sem0b)

            @pl.when((nxt < nchunks) & jnp.logical_not(even_nxt))
            def _start_odd():
                start_chunk(base + nxt * CHUNK, 1, sem1, sem1b)

            even_cur = lax.rem(g, 2) == 0

            @pl.when(even_cur)
            def _wait_even():
                wait_chunk(0, sem0, sem0b)

            @pl.when(jnp.logical_not(even_cur))
            def _wait_odd():
                wait_chunk(1, sem1, sem1b)

            slot = lax.rem(g, 2)
            cs = base + g * CHUNK

            def group(v, c):
                r0 = v * GRP
                row0 = cs + r0
                b = bag_of(row0)
                cutn = cut_at(b)
                uniform = cutn >= row0 + GRP

                def fast():
                    for j in range(nj):
                        sl = pl.ds(j * LANES, LANES)
                        vals = [xbuf[slot, r0 + k, sl] for k in range(GRP)]
                        while len(vals) > 1:
                            vals = [
                                vals[2 * i] + vals[2 * i + 1]
                                for i in range(len(vals) // 2)
                            ] + vals[2 * (len(vals) // 2):]
                        plsc.addupdate(acc.at[b, sl], vals[0])

                def slow():
                    def srow(k, cc):
                        bk = bag_of(row0 + k)
                        for j in range(nj):
                            sl = pl.ds(j * LANES, LANES)
                            plsc.addupdate(acc.at[bk, sl],
                                           xbuf[slot, r0 + k, sl])
                        return cc
                    lax.fori_loop(0, GRP, srow, 0)

                lax.cond(uniform, fast, slow)
                return c

            lax.fori_loop(0, CHUNK // GRP, group, 0)
            return carry

        lax.fori_loop(0, nchunks, chunk_body, 0)

        pltpu.sync_copy(acc, out_hbm.at[wid])

    return body(x, scope1d)


def _tc_onehot_sums(x, scope_f32, sc_rows, total, d, nbags, blk):
    nblk = (total - sc_rows) // blk

    def body(scope_ref, x_ref, out_ref):
        k = pl.program_id(0)
        sc = scope_ref[...].astype(jnp.float32)
        up = sc[1:, :]
        lo = sc[:-1, :]
        r = (jax.lax.broadcasted_iota(jnp.int32, (1, blk), 1)
             + (sc_rows + k * blk)).astype(jnp.float32)
        m = ((r < up).astype(jnp.float32) - (r < lo).astype(jnp.float32))
        part = jax.lax.dot_general(
            m, x_ref[...], (((1,), (0,)), ((), ())),
            preferred_element_type=jnp.float32)

        @pl.when(k == 0)
        def _init():
            out_ref[...] = part

        @pl.when(k > 0)
        def _accum():
            out_ref[...] = out_ref[...] + part

    return pl.pallas_call(
        body,
        grid=(nblk,),
        out_shape=jax.ShapeDtypeStruct((nbags, d), jnp.float32),
        in_specs=[
            pl.BlockSpec((nbags + 1, 1), lambda k: (0, 0),
                         memory_space=pltpu.VMEM),
            pl.BlockSpec((blk, d), lambda k: (sc_rows // blk + k, 0)),
        ],
        out_specs=pl.BlockSpec((nbags, d), lambda k: (0, 0)),
    )(scope_f32, x)


def _tc_project(partials, tc_part, w, bias2d, scope2d, flag):
    nbags = partials.shape[1]
    ncls = w.shape[0]

    def body(part_ref, tcp_ref, wt_ref, bias_ref, scope_ref, flag_ref,
             out_ref):
        sums = jnp.sum(part_ref[...], axis=0) + tcp_ref[...]
        sc = scope_ref[...]
        lengths = (sc[1:, :] - sc[:-1, :]).astype(jnp.float32)
        means = sums / lengths
        logits = jax.lax.dot_general(
            means, wt_ref[...], (((1,), (1,)), ((), ())),
            preferred_element_type=jnp.float32) + bias_ref[...]
        mx = jnp.max(logits, axis=1, keepdims=True)
        e = jnp.exp(logits - mx)
        sm = e / jnp.sum(e, axis=1, keepdims=True)
        out_ref[...] = jnp.where(flag_ref[0, 0] == 1, logits, sm)

    return pl.pallas_call(
        body,
        out_shape=jax.ShapeDtypeStruct((nbags, ncls), jnp.float32),
        in_specs=[
            pl.BlockSpec(memory_space=pltpu.VMEM),
            pl.BlockSpec(memory_space=pltpu.VMEM),
            pl.BlockSpec(memory_space=pltpu.VMEM),
            pl.BlockSpec(memory_space=pltpu.VMEM),
            pl.BlockSpec(memory_space=pltpu.VMEM),
            pl.BlockSpec(memory_space=pltpu.SMEM),
        ],
    )(partials, tc_part, w, bias2d, scope2d, flag)


def kernel(x, W, bias, input_scope, is_train):
    total, d = x.shape
    scope = jnp.asarray(input_scope, jnp.int32)
    nbags = scope.shape[0] - 1
    flag = jnp.asarray(is_train, jnp.int32).reshape(1, 1)
    scope2d = scope.reshape(nbags + 1, 1)

    sc_rows = (total * 5) // 16
    tc_part = _tc_onehot_sums(x, scope2d, sc_rows, total, d, nbags, 2048)
    partials = _sc_partial_sums(x, scope, sc_rows, d, nbags)
    return _tc_project(partials, tc_part, W, bias.reshape(1, -1),
                       scope2d, flag)


# speedup vs baseline: 1.0139x; 1.0139x over previous
import functools

import jax
import jax.numpy as jnp
from jax import lax
from jax.experimental import pallas as pl
from jax.experimental.pallas import tpu as pltpu
from jax.experimental.pallas import tpu_sc as plsc

NC = 2
NS = 16
LANES = 16

CHUNK = 64


def _sc_partial_sums(x, scope1d, sc_rows, d, nbags):
    nworkers = NC * NS
    rows_per_worker = sc_rows // nworkers
    nchunks = rows_per_worker // CHUNK
    nj = d // LANES
    mesh = plsc.VectorSubcoreMesh(core_axis_name="c", subcore_axis_name="s")

    @functools.partial(
        pl.kernel,
        out_type=jax.ShapeDtypeStruct((nworkers, nbags, d), jnp.float32),
        mesh=mesh,
        scratch_types=[
            pltpu.VMEM((2, CHUNK, d), jnp.float32),
            pltpu.VMEM((nbags + 8, ), jnp.int32),
            pltpu.VMEM((nbags, d), jnp.float32),
            pltpu.SemaphoreType.DMA,
            pltpu.SemaphoreType.DMA,
        ],
    )
    def body(x_hbm, scope_hbm, out_hbm, xbuf, cutsv, acc, sem0, sem1):
        c = lax.axis_index("c")
        s = lax.axis_index("s")
        wid = c * NS + s
        base = wid * rows_per_worker

        pltpu.make_async_copy(
            x_hbm.at[pl.ds(base, CHUNK)], xbuf.at[0], sem0).start()
        pltpu.sync_copy(scope_hbm, cutsv.at[pl.ds(0, nbags + 1)])

        def zrow(i, carry):
            for j in range(nj):
                acc[i, pl.ds(j * LANES, LANES)] = jnp.zeros(
                    (LANES,), jnp.float32)
            return carry
        lax.fori_loop(0, nbags, zrow, 0)

        def start_chunk(g0, slot_idx, s_a):
            pltpu.make_async_copy(
                x_hbm.at[pl.ds(g0, CHUNK)], xbuf.at[slot_idx], s_a).start()

        def wait_chunk(slot_idx, s_a):
            pltpu.make_async_copy(
                x_hbm.at[pl.ds(base, CHUNK)], xbuf.at[slot_idx], s_a).wait()

        cuts_vec = cutsv[pl.ds(1, nbags)]
        cut_s = [cuts_vec[i] for i in range(nbags)]

        def bag_of(row):
            b = jnp.int32(0)
            for i in range(nbags):
                b = b + jnp.where(cut_s[i] <= row, 1, 0).astype(jnp.int32)
            return b

        def cut_at(b):
            cn = jnp.int32(0)
            for i in range(nbags):
                cn = cn + jnp.where(b == i, cut_s[i], 0).astype(jnp.int32)
            return cn

        GRP = 16

        def chunk_body(g, carry):
            nxt = g + 1
            even_nxt = lax.rem(nxt, 2) == 0

            @pl.when((nxt < nchunks) & even_nxt)
            def _start_even():
                start_chunk(base + nxt * CHUNK, 0, sem0)

            @pl.when((nxt < nchunks) & jnp.logical_not(even_nxt))
            def _start_odd():
                start_chunk(base + nxt * CHUNK, 1, sem1)

            even_cur = lax.rem(g, 2) == 0

            @pl.when(even_cur)
            def _wait_even():
                wait_chunk(0, sem0)

            @pl.when(jnp.logical_not(even_cur))
            def _wait_odd():
                wait_chunk(1, sem1)

            slot = lax.rem(g, 2)
            cs = base + g * CHUNK

            def group(v, c):
                r0 = v * GRP
                row0 = cs + r0
                b = bag_of(row0)
                cutn = cut_at(b)
                uniform = cutn >= row0 + GRP

                def fast():
                    for j in range(nj):
                        sl = pl.ds(j * LANES, LANES)
                        vals = [xbuf[slot, r0 + k, sl] for k in range(GRP)]
                        while len(vals) > 1:
                            vals = [
                                vals[2 * i] + vals[2 * i + 1]
                                for i in range(len(vals) // 2)
                            ] + vals[2 * (len(vals) // 2):]
                        plsc.addupdate(acc.at[b, sl], vals[0])

                def slow():
                    def srow(k, cc):
                        bk = bag_of(row0 + k)
                        for j in range(nj):
                            sl = pl.ds(j * LANES, LANES)
                            plsc.addupdate(acc.at[bk, sl],
                                           xbuf[slot, r0 + k, sl])
                        return cc
                    lax.fori_loop(0, GRP, srow, 0)

                lax.cond(uniform, fast, slow)
                return c

            lax.fori_loop(0, CHUNK // GRP, group, 0)
            return carry

        lax.fori_loop(0, nchunks, chunk_body, 0)

        pltpu.sync_copy(acc, out_hbm.at[wid])

    return body(x, scope1d)


def _tc_onehot_sums(x, scope_f32, sc_rows, total, d, nbags, blk):
    nblk = (total - sc_rows) // blk

    def body(scope_ref, x_ref, out_ref):
        k = pl.program_id(0)
        sc = scope_ref[...].astype(jnp.float32)
        up = sc[1:, :]
        lo = sc[:-1, :]
        r = (jax.lax.broadcasted_iota(jnp.int32, (1, blk), 1)
             + (sc_rows + k * blk)).astype(jnp.float32)
        m = ((r < up).astype(jnp.float32) - (r < lo).astype(jnp.float32))
        part = jax.lax.dot_general(
            m, x_ref[...], (((1,), (0,)), ((), ())),
            preferred_element_type=jnp.float32)

        @pl.when(k == 0)
        def _init():
            out_ref[...] = part

        @pl.when(k > 0)
        def _accum():
            out_ref[...] = out_ref[...] + part

    return pl.pallas_call(
        body,
        grid=(nblk,),
        out_shape=jax.ShapeDtypeStruct((nbags, d), jnp.float32),
        in_specs=[
            pl.BlockSpec((nbags + 1, 1), lambda k: (0, 0),
                         memory_space=pltpu.VMEM),
            pl.BlockSpec((blk, d), lambda k: (sc_rows // blk + k, 0)),
        ],
        out_specs=pl.BlockSpec((nbags, d), lambda k: (0, 0)),
    )(scope_f32, x)


def _tc_project(partials, tc_part, w, bias2d, scope2d, flag):
    nbags = partials.shape[1]
    ncls = w.shape[0]

    def body(part_ref, tcp_ref, wt_ref, bias_ref, scope_ref, flag_ref,
             out_ref):
        sums = jnp.sum(part_ref[...], axis=0) + tcp_ref[...]
        sc = scope_ref[...]
        lengths = (sc[1:, :] - sc[:-1, :]).astype(jnp.float32)
        means = sums / lengths
        logits = jax.lax.dot_general(
            means, wt_ref[...], (((1,), (1,)), ((), ())),
            preferred_element_type=jnp.float32) + bias_ref[...]
        mx = jnp.max(logits, axis=1, keepdims=True)
        e = jnp.exp(logits - mx)
        sm = e / jnp.sum(e, axis=1, keepdims=True)
        out_ref[...] = jnp.where(flag_ref[0, 0] == 1, logits, sm)

    return pl.pallas_call(
        body,
        out_shape=jax.ShapeDtypeStruct((nbags, ncls), jnp.float32),
        in_specs=[
            pl.BlockSpec(memory_space=pltpu.VMEM),
            pl.BlockSpec(memory_space=pltpu.VMEM),
            pl.BlockSpec(memory_space=pltpu.VMEM),
            pl.BlockSpec(memory_space=pltpu.VMEM),
            pl.BlockSpec(memory_space=pltpu.VMEM),
            pl.BlockSpec(memory_space=pltpu.SMEM),
        ],
    )(partials, tc_part, w, bias2d, scope2d, flag)


def kernel(x, W, bias, input_scope, is_train):
    total, d = x.shape
    scope = jnp.asarray(input_scope, jnp.int32)
    nbags = scope.shape[0] - 1
    flag = jnp.asarray(is_train, jnp.int32).reshape(1, 1)
    scope2d = scope.reshape(nbags + 1, 1)

    sc_rows = (total * 5) // 16
    tc_part = _tc_onehot_sums(x, scope2d, sc_rows, total, d, nbags, 2048)
    partials = _sc_partial_sums(x, scope, sc_rows, d, nbags)
    return _tc_project(partials, tc_part, W, bias.reshape(1, -1),
                       scope2d, flag)
